# baseline (speedup 1.0000x reference)
import functools

import jax
import jax.numpy as jnp
from jax import lax
from jax.experimental import pallas as pl
from jax.experimental.pallas import tpu as pltpu
from jax.experimental.pallas import tpu_sc as plsc

N = 10000
E = 320000
NODE_F = 128
EDGE_F = 16
HID = 64
B = 10
NPG = 1000

NPART = 1
EH = E // NPART

NC = 2
NS = 16
NW = NC * NS
KH = 1000
RPT = N // NS

_F32 = jnp.float32


def _ln(v):
    m = jnp.mean(v, axis=-1, keepdims=True)
    var = jnp.mean((v - m) ** 2, axis=-1, keepdims=True)
    return (v - m) / jnp.sqrt(var + 1e-5)


def _mm(a, b):
    return jnp.dot(a, b, preferred_element_type=_F32,
                   precision=lax.Precision.HIGHEST)


def _gather_body(ne, kg, hA, hB, srcI, dstI, chain, g1,
                 sidx, didx, r1, s1, s2):
    del chain
    cid = lax.axis_index("c")
    sid = lax.axis_index("s")
    wid = sid * NC + cid
    epw = ne // NW
    base = wid * epw

    def chunk(c, carry):
        off = pl.multiple_of(base + c * kg, 8)
        pltpu.sync_copy(srcI.at[pl.ds(off, kg)], sidx)
        pltpu.sync_copy(dstI.at[pl.ds(off, kg)], didx)
        pltpu.async_copy(hA.at[sidx], r1, s1).wait()
        pltpu.async_copy(hB.at[didx], r1, s2, add=True).wait()
        pltpu.sync_copy(r1, g1.at[pl.ds(off, kg)])
        return carry

    lax.fori_loop(0, epw // kg, chunk, 0)


@functools.cache
def _sc_gather(ne, kg):
    return pl.kernel(
        functools.partial(_gather_body, ne, kg),
        out_type=[jax.ShapeDtypeStruct((ne, HID), _F32)],
        mesh=plsc.VectorSubcoreMesh(
            core_axis_name="c", subcore_axis_name="s",
            num_cores=NC, num_subcores=NS,
        ),
        scratch_types=[
            pltpu.VMEM((kg,), jnp.int32),
            pltpu.VMEM((kg,), jnp.int32),
            pltpu.VMEM((kg, HID), _F32),
            pltpu.SemaphoreType.DMA,
            pltpu.SemaphoreType.DMA,
        ],
        compiler_params=pltpu.CompilerParams(use_tc_tiling_on_sc=False),
    )


def _scatter_body(with_deg, ne, ks, eNew, dstI, init64, init16, chain,
                  accp, degp, didx, rows, ones, accS, degS):
    del chain
    cid = lax.axis_index("c")
    sid = lax.axis_index("s")
    wid = sid * NC + cid
    epw = ne // NW
    base = wid * epw
    rbase = sid * RPT

    pltpu.sync_copy(init64.at[pl.ds(cid * N + rbase, RPT)],
                    accS.at[pl.ds(rbase, RPT)])
    if with_deg:
        pltpu.sync_copy(init16.at[pl.ds(cid * N + rbase, RPT)],
                        degS.at[pl.ds(rbase, RPT)])
        onevec = jnp.where(lax.iota(jnp.int32, 16) == 0,
                           jnp.full((16,), 1.0, _F32),
                           jnp.full((16,), 0.0, _F32))

        def fill(i, carry):
            ones[i, :] = onevec
            return carry

        lax.fori_loop(0, ks, fill, 0)
    plsc.subcore_barrier()

    def chunk(c, carry):
        off = pl.multiple_of(base + c * ks, 8)
        pltpu.sync_copy(dstI.at[pl.ds(off, ks)], didx)
        pltpu.sync_copy(eNew.at[pl.ds(off, ks)], rows)
        pltpu.sync_copy(rows, accS.at[didx], add=True)
        if with_deg:
            pltpu.sync_copy(ones, degS.at[didx], add=True)
        return carry

    lax.fori_loop(0, epw // ks, chunk, 0)
    plsc.subcore_barrier()

    pltpu.sync_copy(accS.at[pl.ds(rbase, RPT)],
                    accp.at[pl.ds(cid * N + rbase, RPT)])
    if with_deg:
        pltpu.sync_copy(degS.at[pl.ds(rbase, RPT)],
                        degp.at[pl.ds(cid * N + rbase, RPT)])


@functools.cache
def _make_scatter(with_deg, ne, ks):
    return pl.kernel(
        functools.partial(_scatter_body, with_deg, ne, ks),
        out_type=[
            jax.ShapeDtypeStruct((2 * N, HID), _F32),
            jax.ShapeDtypeStruct((2 * N, 16), _F32),
        ],
        mesh=plsc.VectorSubcoreMesh(
            core_axis_name="c", subcore_axis_name="s",
            num_cores=NC, num_subcores=NS,
        ),
        scratch_types=[
            pltpu.VMEM((ks,), jnp.int32),
            pltpu.VMEM((ks, HID), _F32),
            pltpu.VMEM((ks, 16), _F32),
            pltpu.VMEM_SHARED((N, HID), _F32),
            pltpu.VMEM_SHARED((N, 16), _F32),
        ],
        compiler_params=pltpu.CompilerParams(use_tc_tiling_on_sc=False),
    )


BLKN = 2000
BLKE = 4000
NBEH = EH // BLKE


def _node0_body(x_ref, w_ref, b_ref, a_ref, bw_ref, u_ref, d_ref, nc_ref,
                h_ref, hA_ref, hB_ref, ud_ref, un_ref):
    h = jax.nn.relu(_mm(x_ref[...], w_ref[...]) + b_ref[...])
    h_ref[...] = h
    hA_ref[...] = _mm(h, a_ref[...])
    hB_ref[...] = _mm(h, bw_ref[...])

    @pl.when(pl.program_id(0) == 0)
    def _():
        ud_ref[...] = _mm(u_ref[...], d_ref[...])
        un_ref[...] = _mm(u_ref[...], nc_ref[...])


def _node0(x, wnp, bnp, A1, B1, u0, D1, n1c1):
    full = lambda s: pl.BlockSpec(s, lambda i: (0,) * len(s))
    return pl.pallas_call(
        _node0_body,
        grid=(N // BLKN,),
        in_specs=[
            pl.BlockSpec((BLKN, NODE_F), lambda i: (i, 0)),
            full((NODE_F, HID)), full((1, HID)),
            full((HID, HID)), full((HID, HID)),
            full((B, HID)), full((HID, HID)), full((HID, HID)),
        ],
        out_specs=[pl.BlockSpec((BLKN, HID), lambda i: (i, 0))] * 3
                  + [full((B, HID))] * 2,
        out_shape=[jax.ShapeDtypeStruct((N, HID), _F32)] * 3
                  + [jax.ShapeDtypeStruct((B, HID), _F32)] * 2,
    )(x, wnp, bnp, A1, B1, u0, D1, n1c1)


def _edge_body(first, g1_ref, ep_ref, src_ref, wep_ref, bep_ref,
               c_ref, ud_ref, b1_ref, w2_ref, b2_ref,
               e_ref, mep_ref, epg_ref):
    i = pl.program_id(0)
    srcv = src_ref[0]
    g = srcv // NPG
    oneh = (g == lax.broadcasted_iota(jnp.int32, (BLKE, B), 1)).astype(_F32)
    if first:
        e0 = jax.nn.relu(_mm(ep_ref[...], wep_ref[...]) + bep_ref[...])
    else:
        e0 = ep_ref[...]
    t = (g1_ref[...] + _mm(e0, c_ref[...])
         + _mm(oneh, ud_ref[...]) + b1_ref[...])
    z = jax.nn.relu(t)
    e_new = _ln(_mm(z, w2_ref[...]) + b2_ref[...])
    e_ref[...] = e_new

    @pl.when(i == 0)
    def _():
        mep_ref[...] = jnp.zeros_like(mep_ref)
        epg_ref[...] = jnp.zeros_like(epg_ref)

    mep_ref[...] += lax.dot_general(oneh, e_new, (((0,), (0,)), ((), ())),
                                    preferred_element_type=_F32)
    epg_ref[...] += jnp.sum(oneh, axis=0)[:, None]


def _edge_pass(first, g1, eprev, src3d, wep, bep, C, uD, b1, w2, b2):
    full = lambda s: pl.BlockSpec(s, lambda i: (0,) * len(s))
    ep_spec = (pl.BlockSpec((BLKE, EDGE_F), lambda i: (i, 0)) if first
               else pl.BlockSpec((BLKE, HID), lambda i: (i, 0)))
    return pl.pallas_call(
        functools.partial(_edge_body, first),
        grid=(NBEH,),
        in_specs=[
            pl.BlockSpec((BLKE, HID), lambda i: (i, 0)),
            ep_spec,
            pl.BlockSpec((1, BLKE, 1), lambda i: (i, 0, 0)),
            full((EDGE_F, HID)), full((1, HID)),
            full((HID, HID)), full((B, HID)), full((1, HID)),
            full((HID, HID)), full((1, HID)),
        ],
        out_specs=[
            pl.BlockSpec((BLKE, HID), lambda i: (i, 0)),
            full((B, HID)),
            full((B, 1)),
        ],
        out_shape=[
            jax.ShapeDtypeStruct((EH, HID), _F32),
            jax.ShapeDtypeStruct((B, HID), _F32),
            jax.ShapeDtypeStruct((B, 1), _F32),
        ],
    )(g1, eprev, src3d, wep, bep, C, uD, b1, w2, b2)


def _node_body(last, h_ref, p00, p01, d00, d01, un_ref,
               n1a_ref, n1b_ref, bn1_ref, n2_ref, bn2_ref,
               wa_ref, wb_ref, ba_ref, bb_ref,
               o1_ref, o2_ref, o3_ref, mx_ref):
    deg = jnp.maximum(d00[:, 0:1] + d01[:, 0:1], 1.0)
    agg = (p00[...] + p01[...]) / deg
    t = (_mm(h_ref[...], n1a_ref[...]) + _mm(agg, n1b_ref[...])
         + un_ref[0] + bn1_ref[...])
    hn = _ln(_mm(jax.nn.relu(t), n2_ref[...]) + bn2_ref[...])
    mx_ref[0, 0, :] = jnp.sum(hn, axis=0) / float(NPG)
    if last:
        q = jax.nn.relu(_mm(hn, wa_ref[...]) + ba_ref[...])
        o1_ref[...] = _mm(q, wb_ref[...]) + bb_ref[...]
        o2_ref[...] = hn
        o3_ref[...] = hn
    else:
        o1_ref[...] = hn
        o2_ref[...] = _mm(hn, wa_ref[...])
        o3_ref[...] = _mm(hn, wb_ref[...])


def _node_pass(last, h, ps, ds, uN, n1a, n1b, bn1, n2, bn2, wa, wb, ba, bb):
    full = lambda s: pl.BlockSpec(s, lambda i: (0,) * len(s))
    nblk = lambda w: pl.BlockSpec((NPG, w), lambda i: (i, 0))
    o1_shape = ((N, 1) if last else (N, HID))
    o1_spec = (pl.BlockSpec((NPG, 1), lambda i: (i, 0)) if last
               else pl.BlockSpec((NPG, HID), lambda i: (i, 0)))
    return pl.pallas_call(
        functools.partial(_node_body, last),
        grid=(B,),
        in_specs=[
            nblk(HID), nblk(HID), nblk(HID),
            nblk(16), nblk(16),
            pl.BlockSpec((1, 1, HID), lambda i: (i, 0, 0)),
            full((HID, HID)), full((HID, HID)), full((1, HID)),
            full((HID, HID)), full((1, HID)),
            full((HID, HID)), full((HID, 1 if last else HID)),
            full((1, HID)), full((1, 1 if last else HID)),
        ],
        out_specs=[
            o1_spec,
            pl.BlockSpec((NPG, HID), lambda i: (i, 0)),
            pl.BlockSpec((NPG, HID), lambda i: (i, 0)),
            pl.BlockSpec((1, 1, HID), lambda i: (i, 0, 0)),
        ],
        out_shape=[
            jax.ShapeDtypeStruct(o1_shape, _F32),
            jax.ShapeDtypeStruct((N, HID), _F32),
            jax.ShapeDtypeStruct((N, HID), _F32),
            jax.ShapeDtypeStruct((B, 1, HID), _F32),
        ],
    )(h, *ps, *ds, uN, n1a, n1b, bn1, n2, bn2, wa, wb, ba, bb)


def _glue0_body(u_ref, d_ref, nc_ref, ud_ref, un_ref):
    ud_ref[...] = _mm(u_ref[...], d_ref[...])
    un_ref[...] = _mm(u_ref[...], nc_ref[...])


def _glue0(u0, D1, n1c1):
    return pl.pallas_call(
        _glue0_body,
        out_shape=[jax.ShapeDtypeStruct((B, HID), _F32)] * 2,
    )(u0, D1, n1c1)


def _glue_body(last, u_ref, mx_ref, mepa_ref, mepb_ref, epga_ref, epgb_ref,
               u1a_ref, u1b_ref, u1c_ref, bu1_ref, u2_ref, bu2_ref,
               wa_ref, wb_ref, ba_ref, bb_ref,
               uo_ref, o1_ref, o2_ref):
    me = ((mepa_ref[...] + mepb_ref[...])
          / jnp.maximum(epga_ref[...] + epgb_ref[...], 1.0))
    t = (_mm(u_ref[...], u1a_ref[...]) + _mm(mx_ref[...], u1b_ref[...])
         + _mm(me, u1c_ref[...]) + bu1_ref[...])
    un = _ln(_mm(jax.nn.relu(t), u2_ref[...]) + bu2_ref[...])
    uo_ref[...] = un
    if last:
        q = jax.nn.relu(_mm(un, wa_ref[...]) + ba_ref[...])
        o1_ref[...] = _mm(q, wb_ref[...]) + bb_ref[...]
        o2_ref[...] = un
    else:
        o1_ref[...] = _mm(un, wa_ref[...])
        o2_ref[...] = _mm(un, wb_ref[...])


def _glue(last, u, mx, mepa, mepb, epga, epgb,
          u1a, u1b, u1c, bu1, u2, bu2, wa, wb, ba, bb):
    o1w = 1 if last else HID
    return pl.pallas_call(
        functools.partial(_glue_body, last),
        out_shape=[
            jax.ShapeDtypeStruct((B, HID), _F32),
            jax.ShapeDtypeStruct((B, o1w), _F32),
            jax.ShapeDtypeStruct((B, HID), _F32),
        ],
    )(u, mx, mepa, mepb, epga, epgb, u1a, u1b, u1c, bu1, u2, bu2,
      wa, wb, ba, bb)


def _head_body(lg_ref, m_ref, a_ref, lp_ref, ent_ref):
    lg = jnp.where(m_ref[...] > 0, lg_ref[...], -jnp.inf)
    mx = jnp.max(lg, axis=1, keepdims=True)
    ex = jnp.exp(lg - mx)
    lse = mx + jnp.log(jnp.sum(ex, axis=1, keepdims=True))
    logp = lg - lse
    ids = lax.broadcasted_iota(jnp.int32, (B, NPG), 1)
    sel = ids == a_ref[...]
    lp_ref[...] = jnp.sum(jnp.where(sel, logp, 0.0), axis=1, keepdims=True)
    pr = jnp.exp(logp)
    ent_ref[...] = -jnp.sum(jnp.where(pr > 0, pr * logp, 0.0),
                            axis=1, keepdims=True)


def _head(logits, maskf, act2d):
    return pl.pallas_call(
        _head_body,
        out_shape=[jax.ShapeDtypeStruct((B, 1), _F32)] * 2,
    )(logits, maskf, act2d)


def _layer(first, hA, hB, eprev_h, srch, dsth, src3dh, z64f, z16f, sc_tok,
           wep, bep, C, uD, b1, w2, b2):
    g0 = _sc_gather(EH, KH)(hA, hB, srch[0], dsth[0], sc_tok)[0]
    eo0 = _edge_pass(first, g0, eprev_h[0], src3dh[0],
                     wep, bep, C, uD, b1, w2, b2)
    sc = _make_scatter(first, EH, KH)
    acc0, deg0 = sc(eo0[0], dsth[0], z64f, z16f, g0[:8])
    ps = [acc0[:N], acc0[N:]]
    degs = [deg0[:N], deg0[N:]]
    return (eo0, eo0), ps, degs, acc0[:8]


def kernel(x, edge_index, edge_attr, batch, mask, actions, params):
    src = edge_index[0]
    dst = edge_index[1]
    srch = [src[p * EH:(p + 1) * EH] for p in range(NPART)]
    dsth = [dst[p * EH:(p + 1) * EH] for p in range(NPART)]
    src3dh = [srch[p].reshape(NBEH, BLKE, 1) for p in range(NPART)]
    eah = [edge_attr[p * EH:(p + 1) * EH] for p in range(NPART)]

    lp1, lp2 = params['layers'][0], params['layers'][1]
    A1, B1, C1, D1 = (lp1['e1'][0][k * HID:(k + 1) * HID] for k in range(4))
    A2, B2, C2, D2 = (lp2['e1'][0][k * HID:(k + 1) * HID] for k in range(4))
    n1a1, n1b1, n1c1 = (lp1['n1'][0][k * HID:(k + 1) * HID] for k in range(3))
    n1a2, n1b2, n1c2 = (lp2['n1'][0][k * HID:(k + 1) * HID] for k in range(3))
    u1a1, u1b1, u1c1 = (lp1['u1'][0][k * HID:(k + 1) * HID] for k in range(3))
    u1a2, u1b2, u1c2 = (lp2['u1'][0][k * HID:(k + 1) * HID] for k in range(3))
    r2 = lambda v: v.reshape(1, -1)

    u0 = jnp.tile(params['init_u'], (B, 1))
    z64f = jnp.zeros((2 * N, HID), _F32)
    z16f = jnp.zeros((2 * N, 16), _F32)

    h0, hA1, hB1, uD1, uN1 = _node0(x, params['node_proc'][0],
                                    r2(params['node_proc'][1]),
                                    A1, B1, u0, D1, n1c1)

    eo1, ps1, dg1, tok1 = _layer(True, hA1, hB1, eah, srch, dsth, src3dh,
                                 z64f, z16f, z64f[:8],
                                 params['edge_proc'][0],
                                 r2(params['edge_proc'][1]),
                                 C1, uD1, r2(lp1['e1'][1]),
                                 lp1['e2'][0], r2(lp1['e2'][1]))
    h1, hA2, hB2, mx1 = _node_pass(False, h0, ps1, dg1,
                                   uN1.reshape(B, 1, HID),
                                   n1a1, n1b1, r2(lp1['n1'][1]),
                                   lp1['n2'][0], r2(lp1['n2'][1]),
                                   A2, B2, r2(lp1['n2'][1]),
                                   r2(lp1['n2'][1]))
    u1, uD2, uN2 = _glue(False, u0, mx1.reshape(B, HID),
                         eo1[0][1], eo1[1][1], eo1[0][2], eo1[1][2],
                         u1a1, u1b1, u1c1, r2(lp1['u1'][1]),
                         lp1['u2'][0], r2(lp1['u2'][1]),
                         D2, n1c2, r2(lp1['u2'][1]), r2(lp1['u2'][1]))

    e1h = [eo1[p][0] for p in range(NPART)]
    eo2, ps2, _dg2, _tok2 = _layer(False, hA2, hB2, e1h, srch, dsth, src3dh,
                                   z64f, z16f, tok1,
                                   params['edge_proc'][0],
                                   r2(params['edge_proc'][1]),
                                   C2, uD2, r2(lp2['e1'][1]),
                                   lp2['e2'][0], r2(lp2['e2'][1]))
    logits_col, _ph1, _ph2, mx2 = _node_pass(
        True, h1, ps2, dg1, uN2.reshape(B, 1, HID),
        n1a2, n1b2, r2(lp2['n1'][1]), lp2['n2'][0], r2(lp2['n2'][1]),
        params['a1'][0], params['a2'][0], r2(params['a1'][1]),
        r2(params['a2'][1]))
    value, _vph = _glue(True, u1, mx2.reshape(B, HID),
                        eo2[0][1], eo2[1][1], eo2[0][2], eo2[1][2],
                        u1a2, u1b2, u1c2, r2(lp2['u1'][1]),
                        lp2['u2'][0], r2(lp2['u2'][1]),
                        params['c1'][0], params['c2'][0],
                        r2(params['c1'][1]), r2(params['c2'][1]))[1:]

    logits = logits_col.reshape(B, NPG)
    lp, ent = _head(logits, mask.astype(_F32), actions.reshape(B, 1))
    return (actions, lp.reshape(B), ent.reshape(B), value)


# speedup vs baseline: 1.7640x; 1.7640x over previous
import functools

import jax
import jax.numpy as jnp
from jax import lax
from jax.experimental import pallas as pl
from jax.experimental.pallas import tpu as pltpu
from jax.experimental.pallas import tpu_sc as plsc

N = 10000
E = 320000
NODE_F = 128
EDGE_F = 16
HID = 64
B = 10
NPG = 1000

NPART = 1
EH = E // NPART

NC = 2
NS = 16
NW = NC * NS
KH = 1000
RPT = N // NS

_F32 = jnp.float32


def _ln(v):
    m = jnp.mean(v, axis=-1, keepdims=True)
    var = jnp.mean((v - m) ** 2, axis=-1, keepdims=True)
    return (v - m) / jnp.sqrt(var + 1e-5)


def _mm(a, b):
    return jnp.dot(a, b, preferred_element_type=_F32)


def _gather_body(ne, kg, hA, hB, srcI, dstI, chain, g1,
                 sidx, didx, r1, s1, s2):
    del chain
    cid = lax.axis_index("c")
    sid = lax.axis_index("s")
    wid = sid * NC + cid
    epw = ne // NW
    base = wid * epw

    def chunk(c, carry):
        off = pl.multiple_of(base + c * kg, 8)
        pltpu.sync_copy(srcI.at[pl.ds(off, kg)], sidx)
        pltpu.sync_copy(dstI.at[pl.ds(off, kg)], didx)
        pltpu.async_copy(hA.at[sidx], r1, s1).wait()
        pltpu.async_copy(hB.at[didx], r1, s2, add=True).wait()
        pltpu.sync_copy(r1, g1.at[pl.ds(off, kg)])
        return carry

    lax.fori_loop(0, epw // kg, chunk, 0)


@functools.cache
def _sc_gather(ne, kg):
    return pl.kernel(
        functools.partial(_gather_body, ne, kg),
        out_type=[jax.ShapeDtypeStruct((ne, HID), _F32)],
        mesh=plsc.VectorSubcoreMesh(
            core_axis_name="c", subcore_axis_name="s",
            num_cores=NC, num_subcores=NS,
        ),
        scratch_types=[
            pltpu.VMEM((kg,), jnp.int32),
            pltpu.VMEM((kg,), jnp.int32),
            pltpu.VMEM((kg, HID), _F32),
            pltpu.SemaphoreType.DMA,
            pltpu.SemaphoreType.DMA,
        ],
        compiler_params=pltpu.CompilerParams(use_tc_tiling_on_sc=False),
    )


def _scatter_body(with_deg, ne, ks, eNew, dstI, init64, init16, chain,
                  accp, degp, didx, rows, ones, accS, degS):
    del chain
    cid = lax.axis_index("c")
    sid = lax.axis_index("s")
    wid = sid * NC + cid
    epw = ne // NW
    base = wid * epw
    rbase = sid * RPT

    pltpu.sync_copy(init64.at[pl.ds(cid * N + rbase, RPT)],
                    accS.at[pl.ds(rbase, RPT)])
    if with_deg:
        pltpu.sync_copy(init16.at[pl.ds(cid * N + rbase, RPT)],
                        degS.at[pl.ds(rbase, RPT)])
        onevec = jnp.where(lax.iota(jnp.int32, 16) == 0,
                           jnp.full((16,), 1.0, _F32),
                           jnp.full((16,), 0.0, _F32))

        def fill(i, carry):
            ones[i, :] = onevec
            return carry

        lax.fori_loop(0, ks, fill, 0)
    plsc.subcore_barrier()

    def chunk(c, carry):
        off = pl.multiple_of(base + c * ks, 8)
        pltpu.sync_copy(dstI.at[pl.ds(off, ks)], didx)
        pltpu.sync_copy(eNew.at[pl.ds(off, ks)], rows)
        pltpu.sync_copy(rows, accS.at[didx], add=True)
        if with_deg:
            pltpu.sync_copy(ones, degS.at[didx], add=True)
        return carry

    lax.fori_loop(0, epw // ks, chunk, 0)
    plsc.subcore_barrier()

    pltpu.sync_copy(accS.at[pl.ds(rbase, RPT)],
                    accp.at[pl.ds(cid * N + rbase, RPT)])
    if with_deg:
        pltpu.sync_copy(degS.at[pl.ds(rbase, RPT)],
                        degp.at[pl.ds(cid * N + rbase, RPT)])


@functools.cache
def _make_scatter(with_deg, ne, ks):
    return pl.kernel(
        functools.partial(_scatter_body, with_deg, ne, ks),
        out_type=[
            jax.ShapeDtypeStruct((2 * N, HID), _F32),
            jax.ShapeDtypeStruct((2 * N, 16), _F32),
        ],
        mesh=plsc.VectorSubcoreMesh(
            core_axis_name="c", subcore_axis_name="s",
            num_cores=NC, num_subcores=NS,
        ),
        scratch_types=[
            pltpu.VMEM((ks,), jnp.int32),
            pltpu.VMEM((ks, HID), _F32),
            pltpu.VMEM((ks, 16), _F32),
            pltpu.VMEM_SHARED((N, HID), _F32),
            pltpu.VMEM_SHARED((N, 16), _F32),
        ],
        compiler_params=pltpu.CompilerParams(use_tc_tiling_on_sc=False),
    )


BLKN = 2000
BLKE = 4000
NBEH = EH // BLKE


def _node0_body(x_ref, w_ref, b_ref, a_ref, bw_ref, u_ref, d_ref, nc_ref,
                h_ref, hA_ref, hB_ref, ud_ref, un_ref):
    h = jax.nn.relu(_mm(x_ref[...], w_ref[...]) + b_ref[...])
    h_ref[...] = h
    hA_ref[...] = _mm(h, a_ref[...])
    hB_ref[...] = _mm(h, bw_ref[...])

    @pl.when(pl.program_id(0) == 0)
    def _():
        ud_ref[...] = _mm(u_ref[...], d_ref[...])
        un_ref[...] = _mm(u_ref[...], nc_ref[...])


def _node0(x, wnp, bnp, A1, B1, u0, D1, n1c1):
    full = lambda s: pl.BlockSpec(s, lambda i: (0,) * len(s))
    return pl.pallas_call(
        _node0_body,
        grid=(N // BLKN,),
        in_specs=[
            pl.BlockSpec((BLKN, NODE_F), lambda i: (i, 0)),
            full((NODE_F, HID)), full((1, HID)),
            full((HID, HID)), full((HID, HID)),
            full((B, HID)), full((HID, HID)), full((HID, HID)),
        ],
        out_specs=[pl.BlockSpec((BLKN, HID), lambda i: (i, 0))] * 3
                  + [full((B, HID))] * 2,
        out_shape=[jax.ShapeDtypeStruct((N, HID), _F32)] * 3
                  + [jax.ShapeDtypeStruct((B, HID), _F32)] * 2,
    )(x, wnp, bnp, A1, B1, u0, D1, n1c1)


def _edge_body(first, g1_ref, ep_ref, src_ref, wep_ref, bep_ref,
               c_ref, ud_ref, b1_ref, w2_ref, b2_ref,
               e_ref, mep_ref, epg_ref):
    i = pl.program_id(0)
    srcv = src_ref[0]
    g = srcv // NPG
    oneh = (g == lax.broadcasted_iota(jnp.int32, (BLKE, B), 1)).astype(_F32)
    if first:
        e0 = jax.nn.relu(_mm(ep_ref[...], wep_ref[...]) + bep_ref[...])
    else:
        e0 = ep_ref[...]
    t = (g1_ref[...] + _mm(e0, c_ref[...])
         + _mm(oneh, ud_ref[...]) + b1_ref[...])
    z = jax.nn.relu(t)
    e_new = _ln(_mm(z, w2_ref[...]) + b2_ref[...])
    e_ref[...] = e_new

    @pl.when(i == 0)
    def _():
        mep_ref[...] = jnp.zeros_like(mep_ref)
        epg_ref[...] = jnp.zeros_like(epg_ref)

    mep_ref[...] += lax.dot_general(oneh, e_new, (((0,), (0,)), ((), ())),
                                    preferred_element_type=_F32)
    epg_ref[...] += jnp.sum(oneh, axis=0)[:, None]


def _edge_pass(first, g1, eprev, src3d, wep, bep, C, uD, b1, w2, b2):
    full = lambda s: pl.BlockSpec(s, lambda i: (0,) * len(s))
    ep_spec = (pl.BlockSpec((BLKE, EDGE_F), lambda i: (i, 0)) if first
               else pl.BlockSpec((BLKE, HID), lambda i: (i, 0)))
    return pl.pallas_call(
        functools.partial(_edge_body, first),
        grid=(NBEH,),
        in_specs=[
            pl.BlockSpec((BLKE, HID), lambda i: (i, 0)),
            ep_spec,
            pl.BlockSpec((1, BLKE, 1), lambda i: (i, 0, 0)),
            full((EDGE_F, HID)), full((1, HID)),
            full((HID, HID)), full((B, HID)), full((1, HID)),
            full((HID, HID)), full((1, HID)),
        ],
        out_specs=[
            pl.BlockSpec((BLKE, HID), lambda i: (i, 0)),
            full((B, HID)),
            full((B, 1)),
        ],
        out_shape=[
            jax.ShapeDtypeStruct((EH, HID), _F32),
            jax.ShapeDtypeStruct((B, HID), _F32),
            jax.ShapeDtypeStruct((B, 1), _F32),
        ],
    )(g1, eprev, src3d, wep, bep, C, uD, b1, w2, b2)


def _node_body(last, h_ref, p00, p01, d00, d01, un_ref,
               n1a_ref, n1b_ref, bn1_ref, n2_ref, bn2_ref,
               wa_ref, wb_ref, ba_ref, bb_ref,
               o1_ref, o2_ref, o3_ref, mx_ref):
    deg = jnp.maximum(d00[:, 0:1] + d01[:, 0:1], 1.0)
    agg = (p00[...] + p01[...]) / deg
    t = (_mm(h_ref[...], n1a_ref[...]) + _mm(agg, n1b_ref[...])
         + un_ref[0] + bn1_ref[...])
    hn = _ln(_mm(jax.nn.relu(t), n2_ref[...]) + bn2_ref[...])
    mx_ref[0, 0, :] = jnp.sum(hn, axis=0) / float(NPG)
    if last:
        q = jax.nn.relu(_mm(hn, wa_ref[...]) + ba_ref[...])
        o1_ref[...] = _mm(q, wb_ref[...]) + bb_ref[...]
        o2_ref[...] = hn
        o3_ref[...] = hn
    else:
        o1_ref[...] = hn
        o2_ref[...] = _mm(hn, wa_ref[...])
        o3_ref[...] = _mm(hn, wb_ref[...])


def _node_pass(last, h, ps, ds, uN, n1a, n1b, bn1, n2, bn2, wa, wb, ba, bb):
    full = lambda s: pl.BlockSpec(s, lambda i: (0,) * len(s))
    nblk = lambda w: pl.BlockSpec((NPG, w), lambda i: (i, 0))
    o1_shape = ((N, 1) if last else (N, HID))
    o1_spec = (pl.BlockSpec((NPG, 1), lambda i: (i, 0)) if last
               else pl.BlockSpec((NPG, HID), lambda i: (i, 0)))
    return pl.pallas_call(
        functools.partial(_node_body, last),
        grid=(B,),
        in_specs=[
            nblk(HID), nblk(HID), nblk(HID),
            nblk(16), nblk(16),
            pl.BlockSpec((1, 1, HID), lambda i: (i, 0, 0)),
            full((HID, HID)), full((HID, HID)), full((1, HID)),
            full((HID, HID)), full((1, HID)),
            full((HID, HID)), full((HID, 1 if last else HID)),
            full((1, HID)), full((1, 1 if last else HID)),
        ],
        out_specs=[
            o1_spec,
            pl.BlockSpec((NPG, HID), lambda i: (i, 0)),
            pl.BlockSpec((NPG, HID), lambda i: (i, 0)),
            pl.BlockSpec((1, 1, HID), lambda i: (i, 0, 0)),
        ],
        out_shape=[
            jax.ShapeDtypeStruct(o1_shape, _F32),
            jax.ShapeDtypeStruct((N, HID), _F32),
            jax.ShapeDtypeStruct((N, HID), _F32),
            jax.ShapeDtypeStruct((B, 1, HID), _F32),
        ],
    )(h, *ps, *ds, uN, n1a, n1b, bn1, n2, bn2, wa, wb, ba, bb)


def _glue0_body(u_ref, d_ref, nc_ref, ud_ref, un_ref):
    ud_ref[...] = _mm(u_ref[...], d_ref[...])
    un_ref[...] = _mm(u_ref[...], nc_ref[...])


def _glue0(u0, D1, n1c1):
    return pl.pallas_call(
        _glue0_body,
        out_shape=[jax.ShapeDtypeStruct((B, HID), _F32)] * 2,
    )(u0, D1, n1c1)


def _glue_body(last, u_ref, mx_ref, mepa_ref, mepb_ref, epga_ref, epgb_ref,
               u1a_ref, u1b_ref, u1c_ref, bu1_ref, u2_ref, bu2_ref,
               wa_ref, wb_ref, ba_ref, bb_ref,
               uo_ref, o1_ref, o2_ref):
    me = ((mepa_ref[...] + mepb_ref[...])
          / jnp.maximum(epga_ref[...] + epgb_ref[...], 1.0))
    t = (_mm(u_ref[...], u1a_ref[...]) + _mm(mx_ref[...], u1b_ref[...])
         + _mm(me, u1c_ref[...]) + bu1_ref[...])
    un = _ln(_mm(jax.nn.relu(t), u2_ref[...]) + bu2_ref[...])
    uo_ref[...] = un
    if last:
        q = jax.nn.relu(_mm(un, wa_ref[...]) + ba_ref[...])
        o1_ref[...] = _mm(q, wb_ref[...]) + bb_ref[...]
        o2_ref[...] = un
    else:
        o1_ref[...] = _mm(un, wa_ref[...])
        o2_ref[...] = _mm(un, wb_ref[...])


def _glue(last, u, mx, mepa, mepb, epga, epgb,
          u1a, u1b, u1c, bu1, u2, bu2, wa, wb, ba, bb):
    o1w = 1 if last else HID
    return pl.pallas_call(
        functools.partial(_glue_body, last),
        out_shape=[
            jax.ShapeDtypeStruct((B, HID), _F32),
            jax.ShapeDtypeStruct((B, o1w), _F32),
            jax.ShapeDtypeStruct((B, HID), _F32),
        ],
    )(u, mx, mepa, mepb, epga, epgb, u1a, u1b, u1c, bu1, u2, bu2,
      wa, wb, ba, bb)


def _head_body(lg_ref, m_ref, a_ref, lp_ref, ent_ref):
    lg = jnp.where(m_ref[...] > 0, lg_ref[...], -jnp.inf)
    mx = jnp.max(lg, axis=1, keepdims=True)
    ex = jnp.exp(lg - mx)
    lse = mx + jnp.log(jnp.sum(ex, axis=1, keepdims=True))
    logp = lg - lse
    ids = lax.broadcasted_iota(jnp.int32, (B, NPG), 1)
    sel = ids == a_ref[...]
    lp_ref[...] = jnp.sum(jnp.where(sel, logp, 0.0), axis=1, keepdims=True)
    pr = jnp.exp(logp)
    ent_ref[...] = -jnp.sum(jnp.where(pr > 0, pr * logp, 0.0),
                            axis=1, keepdims=True)


def _head(logits, maskf, act2d):
    return pl.pallas_call(
        _head_body,
        out_shape=[jax.ShapeDtypeStruct((B, 1), _F32)] * 2,
    )(logits, maskf, act2d)


def _layer(first, hA, hB, eprev_h, srch, dsth, src3dh, z64f, z16f, sc_tok,
           wep, bep, C, uD, b1, w2, b2):
    g0 = _sc_gather(EH, KH)(hA, hB, srch[0], dsth[0], sc_tok)[0]
    eo0 = _edge_pass(first, g0, eprev_h[0], src3dh[0],
                     wep, bep, C, uD, b1, w2, b2)
    sc = _make_scatter(first, EH, KH)
    acc0, deg0 = sc(eo0[0], dsth[0], z64f, z16f, g0[:8])
    ps = [acc0[:N], acc0[N:]]
    degs = [deg0[:N], deg0[N:]]
    return (eo0, eo0), ps, degs, acc0[:8]


def kernel(x, edge_index, edge_attr, batch, mask, actions, params):
    src = edge_index[0]
    dst = edge_index[1]
    srch = [src[p * EH:(p + 1) * EH] for p in range(NPART)]
    dsth = [dst[p * EH:(p + 1) * EH] for p in range(NPART)]
    src3dh = [srch[p].reshape(NBEH, BLKE, 1) for p in range(NPART)]
    eah = [edge_attr[p * EH:(p + 1) * EH] for p in range(NPART)]

    lp1, lp2 = params['layers'][0], params['layers'][1]
    A1, B1, C1, D1 = (lp1['e1'][0][k * HID:(k + 1) * HID] for k in range(4))
    A2, B2, C2, D2 = (lp2['e1'][0][k * HID:(k + 1) * HID] for k in range(4))
    n1a1, n1b1, n1c1 = (lp1['n1'][0][k * HID:(k + 1) * HID] for k in range(3))
    n1a2, n1b2, n1c2 = (lp2['n1'][0][k * HID:(k + 1) * HID] for k in range(3))
    u1a1, u1b1, u1c1 = (lp1['u1'][0][k * HID:(k + 1) * HID] for k in range(3))
    u1a2, u1b2, u1c2 = (lp2['u1'][0][k * HID:(k + 1) * HID] for k in range(3))
    r2 = lambda v: v.reshape(1, -1)

    u0 = jnp.tile(params['init_u'], (B, 1))
    z64f = jnp.zeros((2 * N, HID), _F32)
    z16f = jnp.zeros((2 * N, 16), _F32)

    h0, hA1, hB1, uD1, uN1 = _node0(x, params['node_proc'][0],
                                    r2(params['node_proc'][1]),
                                    A1, B1, u0, D1, n1c1)

    eo1, ps1, dg1, tok1 = _layer(True, hA1, hB1, eah, srch, dsth, src3dh,
                                 z64f, z16f, z64f[:8],
                                 params['edge_proc'][0],
                                 r2(params['edge_proc'][1]),
                                 C1, uD1, r2(lp1['e1'][1]),
                                 lp1['e2'][0], r2(lp1['e2'][1]))
    h1, hA2, hB2, mx1 = _node_pass(False, h0, ps1, dg1,
                                   uN1.reshape(B, 1, HID),
                                   n1a1, n1b1, r2(lp1['n1'][1]),
                                   lp1['n2'][0], r2(lp1['n2'][1]),
                                   A2, B2, r2(lp1['n2'][1]),
                                   r2(lp1['n2'][1]))
    u1, uD2, uN2 = _glue(False, u0, mx1.reshape(B, HID),
                         eo1[0][1], eo1[1][1], eo1[0][2], eo1[1][2],
                         u1a1, u1b1, u1c1, r2(lp1['u1'][1]),
                         lp1['u2'][0], r2(lp1['u2'][1]),
                         D2, n1c2, r2(lp1['u2'][1]), r2(lp1['u2'][1]))

    e1h = [eo1[p][0] for p in range(NPART)]
    eo2, ps2, _dg2, _tok2 = _layer(False, hA2, hB2, e1h, srch, dsth, src3dh,
                                   z64f, z16f, tok1,
                                   params['edge_proc'][0],
                                   r2(params['edge_proc'][1]),
                                   C2, uD2, r2(lp2['e1'][1]),
                                   lp2['e2'][0], r2(lp2['e2'][1]))
    logits_col, _ph1, _ph2, mx2 = _node_pass(
        True, h1, ps2, dg1, uN2.reshape(B, 1, HID),
        n1a2, n1b2, r2(lp2['n1'][1]), lp2['n2'][0], r2(lp2['n2'][1]),
        params['a1'][0], params['a2'][0], r2(params['a1'][1]),
        r2(params['a2'][1]))
    value, _vph = _glue(True, u1, mx2.reshape(B, HID),
                        eo2[0][1], eo2[1][1], eo2[0][2], eo2[1][2],
                        u1a2, u1b2, u1c2, r2(lp2['u1'][1]),
                        lp2['u2'][0], r2(lp2['u2'][1]),
                        params['c1'][0], params['c2'][0],
                        r2(params['c1'][1]), r2(params['c2'][1]))[1:]

    logits = logits_col.reshape(B, NPG)
    lp, ent = _head(logits, mask.astype(_F32), actions.reshape(B, 1))
    return (actions, lp.reshape(B), ent.reshape(B), value)
